# Initial kernel scaffold; baseline (speedup 1.0000x reference)
import jax
import jax.numpy as jnp
from jax.experimental import pallas as pl


def kernel(x, table):
    raise NotImplementedError("write your pallas kernel here")


# speedup vs baseline: 4.8058x; 4.8058x over previous
import functools

import jax
import jax.numpy as jnp
from jax import lax
from jax.experimental import pallas as pl
from jax.experimental.pallas import tpu as pltpu
from jax.experimental.pallas import tpu_sc as plsc

BATCH = 16384
HIST_LEN = 200
EMBED_DIM = 32

NUM_WORKERS = 32
CHUNK = 1024


def _gather_kernel(idx_hbm, table_hbm, out_hbm, idx_v, rows_v, sem):
    nc = 2
    wid = lax.axis_index("s") * nc + lax.axis_index("c")
    per_w = (BATCH * HIST_LEN) // NUM_WORKERS
    n_chunks = per_w // CHUNK
    base_w = wid * per_w

    def body(c, carry):
        base = base_w + c * CHUNK
        pltpu.sync_copy(idx_hbm.at[pl.ds(base, CHUNK)], idx_v)
        pltpu.async_copy(table_hbm.at[idx_v], rows_v, sem).wait()
        pltpu.sync_copy(rows_v, out_hbm.at[pl.ds(base, CHUNK)])
        return carry

    lax.fori_loop(0, n_chunks, body, 0)


@jax.jit
def _embedding_lookup(idx, table):
    mesh = plsc.VectorSubcoreMesh(core_axis_name="c", subcore_axis_name="s")
    total = BATCH * HIST_LEN
    return pl.kernel(
        _gather_kernel,
        mesh=mesh,
        out_type=jax.ShapeDtypeStruct((total, EMBED_DIM), jnp.float32),
        scratch_types=[
            pltpu.VMEM((CHUNK,), jnp.int32),
            pltpu.VMEM((CHUNK, EMBED_DIM), jnp.float32),
            pltpu.SemaphoreType.DMA,
        ],
        compiler_params=pltpu.CompilerParams(use_tc_tiling_on_sc=False),
    )(idx, table)


def kernel(x, table):
    idx = x.reshape(-1).astype(jnp.int32)
    out = _embedding_lookup(idx, table)
    return out.reshape(x.shape + (EMBED_DIM,))


# speedup vs baseline: 5.0304x; 1.0467x over previous
import functools

import jax
import jax.numpy as jnp
from jax import lax
from jax.experimental import pallas as pl
from jax.experimental.pallas import tpu as pltpu
from jax.experimental.pallas import tpu_sc as plsc

BATCH = 16384
HIST_LEN = 200
EMBED_DIM = 32
TOTAL = BATCH * HIST_LEN

NUM_WORKERS = 32
CHUNK = 1024


def _gather_kernel(idx_hbm, table_hbm, out_hbm,
                   idx0, idx1, rows0, rows1,
                   sem_i0, sem_i1, sem_g0, sem_g1, sem_s0, sem_s1):
    nc = 2
    wid = lax.axis_index("s") * nc + lax.axis_index("c")
    per_w = TOTAL // NUM_WORKERS
    n_chunks = per_w // CHUNK
    base_w = wid * per_w
    last_base = TOTAL - CHUNK

    def start_idx(c, idx_v, sem):
        off = jnp.minimum(base_w + c * CHUNK, last_base)
        pltpu.make_async_copy(idx_hbm.at[pl.ds(off, CHUNK)], idx_v, sem).start()

    def wait_idx(idx_v, sem):
        pltpu.make_async_copy(idx_hbm.at[pl.ds(0, CHUNK)], idx_v, sem).wait()

    def wait_store(rows_v, sem):
        pltpu.make_async_copy(rows_v, out_hbm.at[pl.ds(0, CHUNK)], sem).wait()

    def do_chunk(c, idx_v, rows_v, sem_i, sem_g, sem_s, first):
        wait_idx(idx_v, sem_i)
        if not first:
            wait_store(rows_v, sem_s)
        g = pltpu.make_async_copy(table_hbm.at[idx_v], rows_v, sem_g)
        g.start()
        g.wait()
        pltpu.make_async_copy(
            rows_v, out_hbm.at[pl.ds(base_w + c * CHUNK, CHUNK)], sem_s
        ).start()
        start_idx(c + 2, idx_v, sem_i)

    start_idx(0, idx0, sem_i0)
    start_idx(1, idx1, sem_i1)
    do_chunk(0, idx0, rows0, sem_i0, sem_g0, sem_s0, True)
    do_chunk(1, idx1, rows1, sem_i1, sem_g1, sem_s1, True)

    def body(p, carry):
        c = 2 * p
        do_chunk(c, idx0, rows0, sem_i0, sem_g0, sem_s0, False)
        do_chunk(c + 1, idx1, rows1, sem_i1, sem_g1, sem_s1, False)
        return carry

    lax.fori_loop(1, n_chunks // 2, body, 0)

    wait_store(rows0, sem_s0)
    wait_store(rows1, sem_s1)
    wait_idx(idx0, sem_i0)
    wait_idx(idx1, sem_i1)


@jax.jit
def _embedding_lookup(idx, table):
    mesh = plsc.VectorSubcoreMesh(core_axis_name="c", subcore_axis_name="s")
    return pl.kernel(
        _gather_kernel,
        mesh=mesh,
        out_type=jax.ShapeDtypeStruct((TOTAL, EMBED_DIM), jnp.float32),
        scratch_types=[
            pltpu.VMEM((CHUNK,), jnp.int32),
            pltpu.VMEM((CHUNK,), jnp.int32),
            pltpu.VMEM((CHUNK, EMBED_DIM), jnp.float32),
            pltpu.VMEM((CHUNK, EMBED_DIM), jnp.float32),
            pltpu.SemaphoreType.DMA,
            pltpu.SemaphoreType.DMA,
            pltpu.SemaphoreType.DMA,
            pltpu.SemaphoreType.DMA,
            pltpu.SemaphoreType.DMA,
            pltpu.SemaphoreType.DMA,
        ],
        compiler_params=pltpu.CompilerParams(use_tc_tiling_on_sc=False),
    )(idx, table)


def kernel(x, table):
    idx = x.reshape(-1).astype(jnp.int32)
    out = _embedding_lookup(idx, table)
    return out.reshape(x.shape + (EMBED_DIM,))
